# baseline (speedup 1.0000x reference)
import math

import jax
import jax.numpy as jnp
from jax.experimental import pallas as pl
from jax.experimental.pallas import tpu as pltpu

N = 3200000
NUM_SEGMENTS = 100000
EPS = 1e-05
PI = math.pi
TWO_PI = 2.0 * math.pi
INV_TWO_PI = 1.0 / TWO_PI

_GRID = 75
_BLK = (3 * N) // _GRID


def _dense_body(t3_ref, f0_ref, v0_ref, evc_ref, erc_ref, ft_ref, vt_ref, rt_ref):
    tt = 2.0 * t3_ref[...]
    e = jnp.exp(-tt)
    v0i = TWO_PI * v0_ref[...]
    f0i = TWO_PI * (jnp.remainder(f0_ref[...] + 0.5, 1.0) - 0.5)
    sigma_v = jnp.sqrt(jnp.clip(1.0 - e * e, EPS, None))
    v_t = e * v0i + sigma_v * evc_ref[...]
    coeff = (1.0 - e) / (1.0 + e)
    mu = coeff * (v_t + v0i)
    mu = jnp.remainder(mu + PI, TWO_PI) - PI
    sig_r = jnp.sqrt(jnp.clip(2.0 * tt + 8.0 / (1.0 + jnp.exp(tt)) - 4.0, EPS, None))
    r_t = jnp.remainder(mu + sig_r * erc_ref[...] + PI, TWO_PI) - PI
    f_t = jnp.remainder(f0i + r_t + PI, TWO_PI) - PI
    ft_ref[...] = f_t * INV_TWO_PI
    vt_ref[...] = v_t * INV_TWO_PI
    rt_ref[...] = r_t * INV_TWO_PI


def _dense(t3, f0f, v0f, evcf, ercf):
    spec = pl.BlockSpec((_BLK,), lambda i: (i,))
    out = pl.pallas_call(
        _dense_body,
        grid=(_GRID,),
        in_specs=[spec] * 5,
        out_specs=[spec] * 3,
        out_shape=[jax.ShapeDtypeStruct((3 * N,), jnp.float32)] * 3,
    )(t3, f0f, v0f, evcf, ercf)
    return out


def _center_xla(x, index):
    seg_sum = jax.ops.segment_sum(x, index, num_segments=NUM_SEGMENTS)
    cnt = jax.ops.segment_sum(jnp.ones((x.shape[0],), dtype=x.dtype), index,
                              num_segments=NUM_SEGMENTS)
    mean = seg_sum / jnp.clip(cnt, 1.0, None)[:, None]
    return x - mean[index]


def kernel(t, f0, index, v0, epsilon_v, epsilon_r):
    evc = _center_xla(epsilon_v, index)
    erc = _center_xla(epsilon_r, index)
    t3 = jnp.broadcast_to(t[:, None], (N, 3)).reshape(-1)
    ft, vt, rt = _dense(t3, f0.reshape(-1), v0.reshape(-1),
                        evc.reshape(-1), erc.reshape(-1))
    return (ft.reshape(N, 3), vt.reshape(N, 3), evc, erc, rt.reshape(N, 3))


# speedup vs baseline: 1.7576x; 1.7576x over previous
import functools
import math

import jax
import jax.numpy as jnp
from jax import lax
from jax.experimental import pallas as pl
from jax.experimental.pallas import tpu as pltpu
from jax.experimental.pallas import tpu_sc as plsc

N = 3200000
S = 100000
EPS = 1e-05
PI = math.pi
TWO_PI = 2.0 * math.pi
INV_TWO_PI = 1.0 / TWO_PI

NW = 32
SUB = 128
NSUB = N // SUB
K = 8
NOUT = NSUB // K
_O_BASE = NOUT // NW
_O_REM = NOUT - _O_BASE * NW
SP = 100096
STRIPE = SP // 16

_mesh = plsc.VectorSubcoreMesh(core_axis_name="c", subcore_axis_name="s")
_sc_params = pltpu.CompilerParams(needs_layout_passes=False,
                                  use_tc_tiling_on_sc=False)


def _worker_span(w):
    base = w * _O_BASE + jnp.minimum(w, _O_REM)
    cnt = jnp.where(w < _O_REM, _O_BASE + 1, _O_BASE)
    return base, cnt


@functools.partial(
    pl.kernel,
    out_type=jax.ShapeDtypeStruct((2, SP, 8), jnp.float32),
    mesh=_mesh,
    scratch_types=[
        pltpu.VMEM((K, SUB), jnp.int32),
        pltpu.VMEM((K * SUB, 8), jnp.float32),
        pltpu.VMEM_SHARED((SP, 8), jnp.float32),
    ],
    compiler_params=_sc_params,
)
def _accum(aug_hbm, idx2d_hbm, zeros_hbm, pout_hbm, idx_buf, aug_buf, sh):
    c = lax.axis_index("c")
    s = lax.axis_index("s")
    w = c * 16 + s
    pltpu.sync_copy(zeros_hbm.at[pl.ds(s * STRIPE, STRIPE)],
                    sh.at[pl.ds(s * STRIPE, STRIPE)])
    plsc.subcore_barrier()
    base, cnt = _worker_span(w)

    @pl.loop(base, base + cnt)
    def _(o):
        j0 = o * K
        pltpu.sync_copy(idx2d_hbm.at[pl.ds(j0, K)], idx_buf)
        pltpu.sync_copy(aug_hbm.at[pl.ds(j0 * SUB, K * SUB)], aug_buf)
        for k in range(K):
            pltpu.sync_copy(aug_buf.at[pl.ds(k * SUB, SUB)],
                            sh.at[idx_buf.at[k]], add=True)

    plsc.subcore_barrier()
    pltpu.sync_copy(sh.at[pl.ds(s * STRIPE, STRIPE)],
                    pout_hbm.at[c].at[pl.ds(s * STRIPE, STRIPE)])


def _merge_body(p_ref, o_ref):
    p = p_ref[...]
    sm = p[0] + p[1]
    cnt = jnp.clip(sm[:, 6:7], 1.0, None)
    o_ref[...] = sm / cnt


def _merge(partials):
    bs = 6256
    return pl.pallas_call(
        _merge_body,
        grid=(SP // bs,),
        in_specs=[pl.BlockSpec((2, bs, 8), lambda i: (0, i, 0))],
        out_specs=pl.BlockSpec((bs, 8), lambda i: (i, 0)),
        out_shape=jax.ShapeDtypeStruct((SP, 8), jnp.float32),
    )(partials)


@functools.partial(
    pl.kernel,
    out_type=[jax.ShapeDtypeStruct((3 * N,), jnp.float32)] * 2,
    mesh=_mesh,
    scratch_types=[
        pltpu.VMEM((K, SUB), jnp.int32),
        pltpu.VMEM((K * 3 * SUB,), jnp.float32),
        pltpu.VMEM((K * 3 * SUB,), jnp.float32),
        pltpu.VMEM((SUB, 8), jnp.float32),
    ],
    compiler_params=_sc_params,
)
def _center(table_hbm, idx2d_hbm, ev_hbm, er_hbm, evc_hbm, erc_hbm,
            idx_buf, ev_buf, er_buf, rows_buf):
    c = lax.axis_index("c")
    s = lax.axis_index("s")
    w = c * 16 + s
    base, cnt = _worker_span(w)
    lanes = lax.iota(jnp.int32, 16)

    @pl.loop(base, base + cnt)
    def _(o):
        j0 = o * K
        f0 = j0 * (3 * SUB)
        pltpu.sync_copy(idx2d_hbm.at[pl.ds(j0, K)], idx_buf)
        pltpu.sync_copy(ev_hbm.at[pl.ds(f0, K * 3 * SUB)], ev_buf)
        pltpu.sync_copy(er_hbm.at[pl.ds(f0, K * 3 * SUB)], er_buf)
        for k in range(K):
            pltpu.sync_copy(table_hbm.at[idx_buf.at[k]], rows_buf)

            @pl.loop(0, 3 * SUB // 16)
            def _(g):
                fi = lanes + g * 16
                r16 = fi // 3
                c3 = fi - r16 * 3
                mv = plsc.load_gather(rows_buf, [r16, c3])
                mr = plsc.load_gather(rows_buf, [r16, c3 + 3])
                fl = k * (3 * SUB) + g * 16
                ev_buf[pl.ds(fl, 16)] = ev_buf[pl.ds(fl, 16)] - mv
                er_buf[pl.ds(fl, 16)] = er_buf[pl.ds(fl, 16)] - mr

        pltpu.sync_copy(ev_buf, evc_hbm.at[pl.ds(f0, K * 3 * SUB)])
        pltpu.sync_copy(er_buf, erc_hbm.at[pl.ds(f0, K * 3 * SUB)])


_GRID = 75
_BLK = (3 * N) // _GRID


def _dense_body(t3_ref, f0_ref, v0_ref, evc_ref, erc_ref, ft_ref, vt_ref, rt_ref):
    tt = 2.0 * t3_ref[...]
    e = jnp.exp(-tt)
    v0i = TWO_PI * v0_ref[...]
    f0i = TWO_PI * (jnp.remainder(f0_ref[...] + 0.5, 1.0) - 0.5)
    sigma_v = jnp.sqrt(jnp.clip(1.0 - e * e, EPS, None))
    v_t = e * v0i + sigma_v * evc_ref[...]
    coeff = (1.0 - e) / (1.0 + e)
    mu = coeff * (v_t + v0i)
    mu = jnp.remainder(mu + PI, TWO_PI) - PI
    sig_r = jnp.sqrt(jnp.clip(2.0 * tt + 8.0 / (1.0 + jnp.exp(tt)) - 4.0, EPS, None))
    r_t = jnp.remainder(mu + sig_r * erc_ref[...] + PI, TWO_PI) - PI
    f_t = jnp.remainder(f0i + r_t + PI, TWO_PI) - PI
    ft_ref[...] = f_t * INV_TWO_PI
    vt_ref[...] = v_t * INV_TWO_PI
    rt_ref[...] = r_t * INV_TWO_PI


def _dense(t3, f0f, v0f, evcf, ercf):
    spec = pl.BlockSpec((_BLK,), lambda i: (i,))
    return pl.pallas_call(
        _dense_body,
        grid=(_GRID,),
        in_specs=[spec] * 5,
        out_specs=[spec] * 3,
        out_shape=[jax.ShapeDtypeStruct((3 * N,), jnp.float32)] * 3,
    )(t3, f0f, v0f, evcf, ercf)


def kernel(t, f0, index, v0, epsilon_v, epsilon_r):
    aug = jnp.concatenate(
        [epsilon_v, epsilon_r,
         jnp.ones((N, 1), jnp.float32), jnp.zeros((N, 1), jnp.float32)], axis=1)
    idx2d = index.reshape(NSUB, SUB)
    zeros = jnp.zeros((SP, 8), jnp.float32)
    partials = _accum(aug, idx2d, zeros)
    table = _merge(partials)
    evc, erc = _center(table, idx2d, epsilon_v.reshape(-1), epsilon_r.reshape(-1))
    t3 = jnp.broadcast_to(t[:, None], (N, 3)).reshape(-1)
    ft, vt, rt = _dense(t3, f0.reshape(-1), v0.reshape(-1), evc, erc)
    return (ft.reshape(N, 3), vt.reshape(N, 3), evc.reshape(N, 3),
            erc.reshape(N, 3), rt.reshape(N, 3))


# speedup vs baseline: 30.4715x; 17.3371x over previous
import functools
import math

import jax
import jax.numpy as jnp
from jax import lax
from jax.experimental import pallas as pl
from jax.experimental.pallas import tpu as pltpu
from jax.experimental.pallas import tpu_sc as plsc

N = 3200000
S = 100000
EPS = 1e-05
PI = math.pi
TWO_PI = 2.0 * math.pi
INV_TWO_PI = 1.0 / TWO_PI

NW = 32
SUB = 128
NSUB = N // SUB
K = 8
CH = K * SUB
NOUT = NSUB // K
_O_BASE = NOUT // NW
_O_REM = NOUT - _O_BASE * NW
SP = 100096
STRIPE = SP // 16
MSTR = SP // 32

_mesh = plsc.VectorSubcoreMesh(core_axis_name="c", subcore_axis_name="s")
_sc_params = pltpu.CompilerParams(needs_layout_passes=False,
                                  use_tc_tiling_on_sc=False)


def _worker_span(w):
    base = w * _O_BASE + jnp.minimum(w, _O_REM)
    cnt = jnp.where(w < _O_REM, _O_BASE + 1, _O_BASE)
    return base, cnt


def _col(v):
    return jnp.full((16,), v, jnp.int32)


@functools.partial(
    pl.kernel,
    out_type=jax.ShapeDtypeStruct((2, SP, 8), jnp.float32),
    mesh=_mesh,
    scratch_types=[
        pltpu.VMEM((CH,), jnp.int32)] + [pltpu.VMEM((CH,), jnp.float32)] * 6 + [
        pltpu.VMEM((SUB, 8), jnp.float32),
        pltpu.VMEM_SHARED((SP, 8), jnp.float32),
    ],
    compiler_params=_sc_params,
)
def _accum(ev0, ev1, ev2, er0, er1, er2, idx_hbm, zeros_hbm, pout_hbm,
           idx_buf, b0, b1, b2, b3, b4, b5, aug_buf, sh):
    c = lax.axis_index("c")
    s = lax.axis_index("s")
    w = c * 16 + s
    lanes = lax.iota(jnp.int32, 16)
    pltpu.sync_copy(zeros_hbm.at[pl.ds(s * STRIPE, STRIPE)],
                    sh.at[pl.ds(s * STRIPE, STRIPE)])

    @pl.loop(0, SUB // 16)
    def _(g):
        r = g * 16 + lanes
        plsc.store_scatter(aug_buf, [r, _col(6)], jnp.full((16,), 1.0, jnp.float32))
        plsc.store_scatter(aug_buf, [r, _col(7)], jnp.full((16,), 0.0, jnp.float32))

    plsc.subcore_barrier()
    base, cnt = _worker_span(w)
    bufs = (b0, b1, b2, b3, b4, b5)

    @pl.loop(base, base + cnt)
    def _(o):
        row0 = o * CH
        pltpu.sync_copy(idx_hbm.at[pl.ds(row0, CH)], idx_buf)
        for b, src in zip(bufs, (ev0, ev1, ev2, er0, er1, er2)):
            pltpu.sync_copy(src.at[pl.ds(row0, CH)], b)
        for k in range(K):
            @pl.loop(0, SUB // 16)
            def _(g):
                r = g * 16 + lanes
                sl = pl.ds(k * SUB + g * 16, 16)
                for col, b in enumerate(bufs):
                    plsc.store_scatter(aug_buf, [r, _col(col)], b[sl])

            pltpu.sync_copy(aug_buf, sh.at[idx_buf.at[pl.ds(k * SUB, SUB)]],
                            add=True)

    plsc.subcore_barrier()
    pltpu.sync_copy(sh.at[pl.ds(s * STRIPE, STRIPE)],
                    pout_hbm.at[c].at[pl.ds(s * STRIPE, STRIPE)])


def _merge_body(p_ref, o_ref):
    p = p_ref[...]
    sm = p[0] + p[1]
    cnt = jnp.clip(sm[:, 6:7], 1.0, None)
    o_ref[...] = sm / cnt


def _merge(partials):
    bs = 3128
    return pl.pallas_call(
        _merge_body,
        grid=(SP // bs,),
        in_specs=[pl.BlockSpec((2, bs, 8), lambda i: (0, i, 0))],
        out_specs=pl.BlockSpec((bs, 8), lambda i: (i, 0)),
        out_shape=jax.ShapeDtypeStruct((SP, 8), jnp.float32),
    )(partials)


@functools.partial(
    pl.kernel,
    out_type=[jax.ShapeDtypeStruct((N,), jnp.float32)] * 6,
    mesh=_mesh,
    scratch_types=[
        pltpu.VMEM((CH,), jnp.int32)] + [pltpu.VMEM((CH,), jnp.float32)] * 6 + [
        pltpu.VMEM((SUB, 8), jnp.float32),
    ],
    compiler_params=_sc_params,
)
def _center(table, idx_hbm, ev0, ev1, ev2, er0, er1, er2,
            oev0, oev1, oev2, oer0, oer1, oer2,
            idx_buf, b0, b1, b2, b3, b4, b5, rows_buf):
    c = lax.axis_index("c")
    s = lax.axis_index("s")
    w = c * 16 + s
    lanes = lax.iota(jnp.int32, 16)
    base, cnt = _worker_span(w)
    bufs = (b0, b1, b2, b3, b4, b5)

    @pl.loop(base, base + cnt)
    def _(o):
        row0 = o * CH
        pltpu.sync_copy(idx_hbm.at[pl.ds(row0, CH)], idx_buf)
        for b, src in zip(bufs, (ev0, ev1, ev2, er0, er1, er2)):
            pltpu.sync_copy(src.at[pl.ds(row0, CH)], b)
        for k in range(K):
            pltpu.sync_copy(table.at[idx_buf.at[pl.ds(k * SUB, SUB)]], rows_buf)

            @pl.loop(0, SUB // 16)
            def _(g):
                r = g * 16 + lanes
                sl = pl.ds(k * SUB + g * 16, 16)
                for col, b in enumerate(bufs):
                    b[sl] = b[sl] - plsc.load_gather(rows_buf, [r, _col(col)])

        for b, dst in zip(bufs, (oev0, oev1, oev2, oer0, oer1, oer2)):
            pltpu.sync_copy(b, dst.at[pl.ds(row0, CH)])


_GRID = 25
_BLK = N // _GRID


def _wrap_pi(x):
    return jnp.remainder(x + PI, TWO_PI) - PI


def _dense_body(t_ref, f00, f01, f02, v00, v01, v02, e0, e1, e2, r0, r1, r2,
                ft0, ft1, ft2, vt0, vt1, vt2, rt0, rt1, rt2):
    tt = 2.0 * t_ref[...]
    e = jnp.exp(-tt)
    sigma_v = jnp.sqrt(jnp.clip(1.0 - e * e, EPS, None))
    coeff = (1.0 - e) / (1.0 + e)
    sig_r = jnp.sqrt(jnp.clip(2.0 * tt + 8.0 / (1.0 + jnp.exp(tt)) - 4.0, EPS, None))
    for f0c, v0c, evc, erc, ftc, vtc, rtc in (
            (f00, v00, e0, r0, ft0, vt0, rt0),
            (f01, v01, e1, r1, ft1, vt1, rt1),
            (f02, v02, e2, r2, ft2, vt2, rt2)):
        v0i = TWO_PI * v0c[...]
        f0i = TWO_PI * (jnp.remainder(f0c[...] + 0.5, 1.0) - 0.5)
        v_t = e * v0i + sigma_v * evc[...]
        mu = _wrap_pi(coeff * (v_t + v0i))
        r_t = _wrap_pi(mu + sig_r * erc[...])
        f_t = _wrap_pi(f0i + r_t)
        ftc[...] = f_t * INV_TWO_PI
        vtc[...] = v_t * INV_TWO_PI
        rtc[...] = r_t * INV_TWO_PI


def _dense(t, planes):
    spec = pl.BlockSpec((_BLK,), lambda i: (i,))
    return pl.pallas_call(
        _dense_body,
        grid=(_GRID,),
        in_specs=[spec] * 13,
        out_specs=[spec] * 9,
        out_shape=[jax.ShapeDtypeStruct((N,), jnp.float32)] * 9,
    )(t, *planes)


def kernel(t, f0, index, v0, epsilon_v, epsilon_r):
    evp = [epsilon_v[:, i] for i in range(3)]
    erp = [epsilon_r[:, i] for i in range(3)]
    f0p = [f0[:, i] for i in range(3)]
    v0p = [v0[:, i] for i in range(3)]
    zeros = jnp.zeros((SP, 8), jnp.float32)
    partials = _accum(*evp, *erp, index, zeros)
    table = _merge(partials)
    cent = _center(table, index, *evp, *erp)
    outs = _dense(t, f0p + v0p + list(cent))
    ft = jnp.stack(outs[0:3], axis=1)
    vt = jnp.stack(outs[3:6], axis=1)
    rt = jnp.stack(outs[6:9], axis=1)
    evc = jnp.stack(cent[0:3], axis=1)
    erc = jnp.stack(cent[3:6], axis=1)
    return (ft, vt, evc, erc, rt)


# speedup vs baseline: 52.8326x; 1.7338x over previous
import functools
import math

import jax
import jax.numpy as jnp
from jax import lax
from jax.experimental import pallas as pl
from jax.experimental.pallas import tpu as pltpu
from jax.experimental.pallas import tpu_sc as plsc

N = 3200000
S = 100000
EPS = 1e-05
PI = math.pi
TWO_PI = 2.0 * math.pi
INV_TWO_PI = 1.0 / TWO_PI

NW = 32
SUB = 128
NSUB = N // SUB
K = 20
CH = K * SUB
NOUT = NSUB // K
_O_BASE = NOUT // NW
_O_REM = NOUT - _O_BASE * NW
SP = 100096
STRIPE = SP // 16
MSTR = SP // 32

_mesh = plsc.VectorSubcoreMesh(core_axis_name="c", subcore_axis_name="s")
_sc_params = pltpu.CompilerParams(needs_layout_passes=False,
                                  use_tc_tiling_on_sc=False)


def _worker_span(w):
    base = w * _O_BASE + jnp.minimum(w, _O_REM)
    cnt = jnp.where(w < _O_REM, _O_BASE + 1, _O_BASE)
    return base, cnt


def _col(v):
    return jnp.full((16,), v, jnp.int32)


@functools.partial(
    pl.kernel,
    out_type=jax.ShapeDtypeStruct((2, SP, 8), jnp.float32),
    mesh=_mesh,
    scratch_types=[
        pltpu.VMEM((CH,), jnp.int32)] + [pltpu.VMEM((CH,), jnp.float32)] * 6 + [
        pltpu.VMEM((CH, 8), jnp.float32),
        pltpu.VMEM_SHARED((SP, 8), jnp.float32),
        pltpu.SemaphoreType.DMA,
        pltpu.SemaphoreType.DMA,
    ],
    compiler_params=_sc_params,
)
def _accum(ev0, ev1, ev2, er0, er1, er2, idx_hbm, zeros_hbm, pout_hbm,
           idx_buf, b0, b1, b2, b3, b4, b5, aug_buf, sh, sem_in, sem_sc):
    c = lax.axis_index("c")
    s = lax.axis_index("s")
    w = c * 16 + s
    lanes = lax.iota(jnp.int32, 16)
    pltpu.sync_copy(zeros_hbm.at[pl.ds(s * STRIPE, STRIPE)],
                    sh.at[pl.ds(s * STRIPE, STRIPE)])

    @pl.loop(0, CH // 16)
    def _(g):
        r = g * 16 + lanes
        plsc.store_scatter(aug_buf, [r, _col(6)], jnp.full((16,), 1.0, jnp.float32))
        plsc.store_scatter(aug_buf, [r, _col(7)], jnp.full((16,), 0.0, jnp.float32))

    plsc.subcore_barrier()
    base, cnt = _worker_span(w)
    bufs = (b0, b1, b2, b3, b4, b5)

    @pl.loop(base, base + cnt)
    def _(o):
        row0 = o * CH
        sl_in = pl.ds(row0, CH)
        descs = [pltpu.async_copy(idx_hbm.at[sl_in], idx_buf, sem_in)]
        for b, src in zip(bufs, (ev0, ev1, ev2, er0, er1, er2)):
            descs.append(pltpu.async_copy(src.at[sl_in], b, sem_in))
        for d in descs:
            d.wait()

        @pl.loop(0, CH // 16)
        def _(g):
            r = g * 16 + lanes
            sl = pl.ds(g * 16, 16)
            for col, b in enumerate(bufs):
                plsc.store_scatter(aug_buf, [r, _col(col)], b[sl])

        scs = []
        for k in range(K):
            sl_k = pl.ds(k * SUB, SUB)
            scs.append(pltpu.async_copy(aug_buf.at[sl_k],
                                        sh.at[idx_buf.at[sl_k]],
                                        sem_sc, add=True))
        for d in scs:
            d.wait()

    plsc.subcore_barrier()
    pltpu.sync_copy(sh.at[pl.ds(s * STRIPE, STRIPE)],
                    pout_hbm.at[c].at[pl.ds(s * STRIPE, STRIPE)])


def _merge_body(p_ref, o_ref):
    p = p_ref[...]
    sm = p[0] + p[1]
    cnt = jnp.clip(sm[:, 6:7], 1.0, None)
    o_ref[...] = sm / cnt


def _merge(partials):
    bs = 3128
    return pl.pallas_call(
        _merge_body,
        grid=(SP // bs,),
        in_specs=[pl.BlockSpec((2, bs, 8), lambda i: (0, i, 0))],
        out_specs=pl.BlockSpec((bs, 8), lambda i: (i, 0)),
        out_shape=jax.ShapeDtypeStruct((SP, 8), jnp.float32),
    )(partials)


@functools.partial(
    pl.kernel,
    out_type=[jax.ShapeDtypeStruct((N,), jnp.float32)] * 6,
    mesh=_mesh,
    scratch_types=[
        pltpu.VMEM((CH,), jnp.int32)] + [pltpu.VMEM((CH,), jnp.float32)] * 6 + [
        pltpu.VMEM((CH, 8), jnp.float32),
        pltpu.SemaphoreType.DMA,
        pltpu.SemaphoreType.DMA,
        pltpu.SemaphoreType.DMA,
    ],
    compiler_params=_sc_params,
)
def _center(table, idx_hbm, ev0, ev1, ev2, er0, er1, er2,
            oev0, oev1, oev2, oer0, oer1, oer2,
            idx_buf, b0, b1, b2, b3, b4, b5, rows_buf, sem_in, sem_g, sem_out):
    c = lax.axis_index("c")
    s = lax.axis_index("s")
    w = c * 16 + s
    lanes = lax.iota(jnp.int32, 16)
    base, cnt = _worker_span(w)
    bufs = (b0, b1, b2, b3, b4, b5)

    @pl.loop(base, base + cnt)
    def _(o):
        row0 = o * CH
        sl_in = pl.ds(row0, CH)
        d_idx = pltpu.async_copy(idx_hbm.at[sl_in], idx_buf, sem_in)
        descs = [pltpu.async_copy(src.at[sl_in], b, sem_in)
                 for b, src in zip(bufs, (ev0, ev1, ev2, er0, er1, er2))]
        d_idx.wait()
        gds = []
        for k in range(K):
            sl_k = pl.ds(k * SUB, SUB)
            gds.append(pltpu.async_copy(table.at[idx_buf.at[sl_k]],
                                        rows_buf.at[sl_k], sem_g))
        for d in descs:
            d.wait()
        for d in gds:
            d.wait()

        @pl.loop(0, CH // 16)
        def _(g):
            r = g * 16 + lanes
            sl = pl.ds(g * 16, 16)
            for col, b in enumerate(bufs):
                b[sl] = b[sl] - plsc.load_gather(rows_buf, [r, _col(col)])

        ods = [pltpu.async_copy(b, dst.at[sl_in], sem_out)
               for b, dst in zip(bufs, (oev0, oev1, oev2, oer0, oer1, oer2))]
        for d in ods:
            d.wait()


_GRID = 25
_BLK = N // _GRID


def _wrap_pi(x):
    return jnp.remainder(x + PI, TWO_PI) - PI


def _dense_body(t_ref, f00, f01, f02, v00, v01, v02, e0, e1, e2, r0, r1, r2,
                ft0, ft1, ft2, vt0, vt1, vt2, rt0, rt1, rt2):
    tt = 2.0 * t_ref[...]
    e = jnp.exp(-tt)
    sigma_v = jnp.sqrt(jnp.clip(1.0 - e * e, EPS, None))
    coeff = (1.0 - e) / (1.0 + e)
    sig_r = jnp.sqrt(jnp.clip(2.0 * tt + 8.0 / (1.0 + jnp.exp(tt)) - 4.0, EPS, None))
    for f0c, v0c, evc, erc, ftc, vtc, rtc in (
            (f00, v00, e0, r0, ft0, vt0, rt0),
            (f01, v01, e1, r1, ft1, vt1, rt1),
            (f02, v02, e2, r2, ft2, vt2, rt2)):
        v0i = TWO_PI * v0c[...]
        f0i = TWO_PI * (jnp.remainder(f0c[...] + 0.5, 1.0) - 0.5)
        v_t = e * v0i + sigma_v * evc[...]
        mu = _wrap_pi(coeff * (v_t + v0i))
        r_t = _wrap_pi(mu + sig_r * erc[...])
        f_t = _wrap_pi(f0i + r_t)
        ftc[...] = f_t * INV_TWO_PI
        vtc[...] = v_t * INV_TWO_PI
        rtc[...] = r_t * INV_TWO_PI


def _dense(t, planes):
    spec = pl.BlockSpec((_BLK,), lambda i: (i,))
    return pl.pallas_call(
        _dense_body,
        grid=(_GRID,),
        in_specs=[spec] * 13,
        out_specs=[spec] * 9,
        out_shape=[jax.ShapeDtypeStruct((N,), jnp.float32)] * 9,
    )(t, *planes)


def kernel(t, f0, index, v0, epsilon_v, epsilon_r):
    evp = [epsilon_v[:, i] for i in range(3)]
    erp = [epsilon_r[:, i] for i in range(3)]
    f0p = [f0[:, i] for i in range(3)]
    v0p = [v0[:, i] for i in range(3)]
    zeros = jnp.zeros((SP, 8), jnp.float32)
    partials = _accum(*evp, *erp, index, zeros)
    table = _merge(partials)
    cent = _center(table, index, *evp, *erp)
    outs = _dense(t, f0p + v0p + list(cent))
    ft = jnp.stack(outs[0:3], axis=1)
    vt = jnp.stack(outs[3:6], axis=1)
    rt = jnp.stack(outs[6:9], axis=1)
    evc = jnp.stack(cent[0:3], axis=1)
    erc = jnp.stack(cent[3:6], axis=1)
    return (ft, vt, evc, erc, rt)


# speedup vs baseline: 58.3247x; 1.1040x over previous
import functools
import math

import jax
import jax.numpy as jnp
from jax import lax
from jax.experimental import pallas as pl
from jax.experimental.pallas import tpu as pltpu
from jax.experimental.pallas import tpu_sc as plsc

N = 3200000
S = 100000
EPS = 1e-05
PI = math.pi
TWO_PI = 2.0 * math.pi
INV_TWO_PI = 1.0 / TWO_PI

NW = 32
SUB = 128
NSUB = N // SUB
K = 10
CH = K * SUB
NOUT = NSUB // K
PAIRS = NOUT // 2
_P_BASE = PAIRS // NW
_P_REM = PAIRS - _P_BASE * NW
SP = 100096
STRIPE = SP // 16
MSTR = SP // 32

_mesh = plsc.VectorSubcoreMesh(core_axis_name="c", subcore_axis_name="s")
_sc_params = pltpu.CompilerParams(needs_layout_passes=False,
                                  use_tc_tiling_on_sc=False)


def _worker_span(w):
    base = w * _P_BASE + jnp.minimum(w, _P_REM)
    cnt = jnp.where(w < _P_REM, _P_BASE + 1, _P_BASE)
    return base, cnt


def _col(v):
    return jnp.full((16,), v, jnp.int32)


@functools.partial(
    pl.kernel,
    out_type=jax.ShapeDtypeStruct((2, SP, 8), jnp.float32),
    mesh=_mesh,
    scratch_types=[
        pltpu.VMEM((CH,), jnp.int32)] * 2 + [pltpu.VMEM((CH,), jnp.float32)] * 12 + [
        pltpu.VMEM((CH, 8), jnp.float32)] * 2 + [
        pltpu.VMEM_SHARED((SP, 8), jnp.float32),
        pltpu.SemaphoreType.DMA,
        pltpu.SemaphoreType.DMA,
        pltpu.SemaphoreType.DMA,
    ],
    compiler_params=_sc_params,
)
def _accum(ev0, ev1, ev2, er0, er1, er2, idx_hbm, zeros_hbm, pout_hbm,
           idx_a, idx_b, a0, a1, a2, a3, a4, a5, c0, c1, c2, c3, c4, c5,
           aug_a, aug_b, sh, sem_a, sem_b, sem_sc):
    c = lax.axis_index("c")
    s = lax.axis_index("s")
    w = c * 16 + s
    lanes = lax.iota(jnp.int32, 16)
    pltpu.sync_copy(zeros_hbm.at[pl.ds(s * STRIPE, STRIPE)],
                    sh.at[pl.ds(s * STRIPE, STRIPE)])
    for aug in (aug_a, aug_b):
        @pl.loop(0, CH // 16)
        def _(g):
            r = g * 16 + lanes
            plsc.store_scatter(aug, [r, _col(6)], jnp.full((16,), 1.0, jnp.float32))
            plsc.store_scatter(aug, [r, _col(7)], jnp.full((16,), 0.0, jnp.float32))

    plsc.subcore_barrier()
    base, cnt = _worker_span(w)
    bufs_a = (a0, a1, a2, a3, a4, a5)
    bufs_b = (c0, c1, c2, c3, c4, c5)
    srcs = (ev0, ev1, ev2, er0, er1, er2)

    def _assemble(bufs, aug):
        @pl.loop(0, CH // 16)
        def _(g):
            r = g * 16 + lanes
            sl = pl.ds(g * 16, 16)
            for col, b in enumerate(bufs):
                plsc.store_scatter(aug, [r, _col(col)], b[sl])

    @pl.loop(base, base + cnt)
    def _(u):
        row_a = (2 * u) * CH
        row_b = row_a + CH
        sl_a = pl.ds(row_a, CH)
        sl_b = pl.ds(row_b, CH)
        da = [pltpu.async_copy(idx_hbm.at[sl_a], idx_a, sem_a)]
        da += [pltpu.async_copy(src.at[sl_a], b, sem_a)
               for b, src in zip(bufs_a, srcs)]
        db = [pltpu.async_copy(idx_hbm.at[sl_b], idx_b, sem_b)]
        db += [pltpu.async_copy(src.at[sl_b], b, sem_b)
               for b, src in zip(bufs_b, srcs)]
        for d in da:
            d.wait()
        _assemble(bufs_a, aug_a)
        scs = []
        for k in range(K):
            sl_k = pl.ds(k * SUB, SUB)
            scs.append(pltpu.async_copy(aug_a.at[sl_k],
                                        sh.at[idx_a.at[sl_k]],
                                        sem_sc, add=True))
        for d in db:
            d.wait()
        _assemble(bufs_b, aug_b)
        for k in range(K):
            sl_k = pl.ds(k * SUB, SUB)
            scs.append(pltpu.async_copy(aug_b.at[sl_k],
                                        sh.at[idx_b.at[sl_k]],
                                        sem_sc, add=True))
        for d in scs:
            d.wait()

    plsc.subcore_barrier()
    pltpu.sync_copy(sh.at[pl.ds(s * STRIPE, STRIPE)],
                    pout_hbm.at[c].at[pl.ds(s * STRIPE, STRIPE)])


def _merge_body(p_ref, o_ref):
    p = p_ref[...]
    sm = p[0] + p[1]
    cnt = jnp.clip(sm[:, 6:7], 1.0, None)
    o_ref[...] = sm / cnt


def _merge(partials):
    bs = 3128
    return pl.pallas_call(
        _merge_body,
        grid=(SP // bs,),
        in_specs=[pl.BlockSpec((2, bs, 8), lambda i: (0, i, 0))],
        out_specs=pl.BlockSpec((bs, 8), lambda i: (i, 0)),
        out_shape=jax.ShapeDtypeStruct((SP, 8), jnp.float32),
    )(partials)


@functools.partial(
    pl.kernel,
    out_type=[jax.ShapeDtypeStruct((N,), jnp.float32)] * 6,
    mesh=_mesh,
    scratch_types=[
        pltpu.VMEM((CH,), jnp.int32)] * 2 + [pltpu.VMEM((CH,), jnp.float32)] * 12 + [
        pltpu.VMEM((CH, 8), jnp.float32)] * 2 + [
        pltpu.SemaphoreType.DMA,
        pltpu.SemaphoreType.DMA,
        pltpu.SemaphoreType.DMA,
        pltpu.SemaphoreType.DMA,
        pltpu.SemaphoreType.DMA,
    ],
    compiler_params=_sc_params,
)
def _center(table, idx_hbm, ev0, ev1, ev2, er0, er1, er2,
            oev0, oev1, oev2, oer0, oer1, oer2,
            idx_a, idx_b, a0, a1, a2, a3, a4, a5, c0, c1, c2, c3, c4, c5,
            rows_a, rows_b, sem_a, sem_b, sem_ga, sem_gb, sem_out):
    c = lax.axis_index("c")
    s = lax.axis_index("s")
    w = c * 16 + s
    lanes = lax.iota(jnp.int32, 16)
    base, cnt = _worker_span(w)
    bufs_a = (a0, a1, a2, a3, a4, a5)
    bufs_b = (c0, c1, c2, c3, c4, c5)
    srcs = (ev0, ev1, ev2, er0, er1, er2)
    outs = (oev0, oev1, oev2, oer0, oer1, oer2)

    def _subtract(bufs, rows):
        @pl.loop(0, CH // 16)
        def _(g):
            r = g * 16 + lanes
            sl = pl.ds(g * 16, 16)
            for col, b in enumerate(bufs):
                b[sl] = b[sl] - plsc.load_gather(rows, [r, _col(col)])

    @pl.loop(base, base + cnt)
    def _(u):
        row_a = (2 * u) * CH
        row_b = row_a + CH
        sl_a = pl.ds(row_a, CH)
        sl_b = pl.ds(row_b, CH)
        dia = pltpu.async_copy(idx_hbm.at[sl_a], idx_a, sem_a)
        dib = pltpu.async_copy(idx_hbm.at[sl_b], idx_b, sem_b)
        da = [pltpu.async_copy(src.at[sl_a], b, sem_a)
              for b, src in zip(bufs_a, srcs)]
        db = [pltpu.async_copy(src.at[sl_b], b, sem_b)
              for b, src in zip(bufs_b, srcs)]
        dia.wait()
        ga = [pltpu.async_copy(table.at[idx_a.at[pl.ds(k * SUB, SUB)]],
                               rows_a.at[pl.ds(k * SUB, SUB)], sem_ga)
              for k in range(K)]
        dib.wait()
        gb = [pltpu.async_copy(table.at[idx_b.at[pl.ds(k * SUB, SUB)]],
                               rows_b.at[pl.ds(k * SUB, SUB)], sem_gb)
              for k in range(K)]
        for d in da:
            d.wait()
        for d in ga:
            d.wait()
        _subtract(bufs_a, rows_a)
        oda = [pltpu.async_copy(b, dst.at[sl_a], sem_out)
               for b, dst in zip(bufs_a, outs)]
        for d in db:
            d.wait()
        for d in gb:
            d.wait()
        _subtract(bufs_b, rows_b)
        odb = [pltpu.async_copy(b, dst.at[sl_b], sem_out)
               for b, dst in zip(bufs_b, outs)]
        for d in oda:
            d.wait()
        for d in odb:
            d.wait()


_GRID = 25
_BLK = N // _GRID


def _wrap_pi(x):
    return jnp.remainder(x + PI, TWO_PI) - PI


def _dense_body(t_ref, f00, f01, f02, v00, v01, v02, e0, e1, e2, r0, r1, r2,
                ft0, ft1, ft2, vt0, vt1, vt2, rt0, rt1, rt2):
    tt = 2.0 * t_ref[...]
    e = jnp.exp(-tt)
    sigma_v = jnp.sqrt(jnp.clip(1.0 - e * e, EPS, None))
    coeff = (1.0 - e) / (1.0 + e)
    sig_r = jnp.sqrt(jnp.clip(2.0 * tt + 8.0 / (1.0 + jnp.exp(tt)) - 4.0, EPS, None))
    for f0c, v0c, evc, erc, ftc, vtc, rtc in (
            (f00, v00, e0, r0, ft0, vt0, rt0),
            (f01, v01, e1, r1, ft1, vt1, rt1),
            (f02, v02, e2, r2, ft2, vt2, rt2)):
        v0i = TWO_PI * v0c[...]
        f0i = TWO_PI * (jnp.remainder(f0c[...] + 0.5, 1.0) - 0.5)
        v_t = e * v0i + sigma_v * evc[...]
        mu = _wrap_pi(coeff * (v_t + v0i))
        r_t = _wrap_pi(mu + sig_r * erc[...])
        f_t = _wrap_pi(f0i + r_t)
        ftc[...] = f_t * INV_TWO_PI
        vtc[...] = v_t * INV_TWO_PI
        rtc[...] = r_t * INV_TWO_PI


def _dense(t, planes):
    spec = pl.BlockSpec((_BLK,), lambda i: (i,))
    return pl.pallas_call(
        _dense_body,
        grid=(_GRID,),
        in_specs=[spec] * 13,
        out_specs=[spec] * 9,
        out_shape=[jax.ShapeDtypeStruct((N,), jnp.float32)] * 9,
    )(t, *planes)


def kernel(t, f0, index, v0, epsilon_v, epsilon_r):
    evp = [epsilon_v[:, i] for i in range(3)]
    erp = [epsilon_r[:, i] for i in range(3)]
    f0p = [f0[:, i] for i in range(3)]
    v0p = [v0[:, i] for i in range(3)]
    zeros = jnp.zeros((SP, 8), jnp.float32)
    partials = _accum(*evp, *erp, index, zeros)
    table = _merge(partials)
    cent = _center(table, index, *evp, *erp)
    outs = _dense(t, f0p + v0p + list(cent))
    ft = jnp.stack(outs[0:3], axis=1)
    vt = jnp.stack(outs[3:6], axis=1)
    rt = jnp.stack(outs[6:9], axis=1)
    evc = jnp.stack(cent[0:3], axis=1)
    erc = jnp.stack(cent[3:6], axis=1)
    return (ft, vt, evc, erc, rt)


# speedup vs baseline: 59.1969x; 1.0150x over previous
import functools
import math

import jax
import jax.numpy as jnp
from jax import lax
from jax.experimental import pallas as pl
from jax.experimental.pallas import tpu as pltpu
from jax.experimental.pallas import tpu_sc as plsc

N = 3200000
S = 100000
EPS = 1e-05
PI = math.pi
TWO_PI = 2.0 * math.pi
INV_TWO_PI = 1.0 / TWO_PI

NW = 32
SUB = 128
NSUB = N // SUB
K = 20
CH = K * SUB
NOUT = NSUB // K
PAIRS = NOUT // 2
_P_BASE = PAIRS // NW
_P_REM = PAIRS - _P_BASE * NW
SP = 100096
STRIPE = SP // 16
MSTR = SP // 32

_mesh = plsc.VectorSubcoreMesh(core_axis_name="c", subcore_axis_name="s")
_sc_params = pltpu.CompilerParams(needs_layout_passes=False,
                                  use_tc_tiling_on_sc=False)


def _worker_span(w):
    base = w * _P_BASE + jnp.minimum(w, _P_REM)
    cnt = jnp.where(w < _P_REM, _P_BASE + 1, _P_BASE)
    return base, cnt


def _col(v):
    return jnp.full((16,), v, jnp.int32)


@functools.partial(
    pl.kernel,
    out_type=jax.ShapeDtypeStruct((2, SP, 8), jnp.float32),
    mesh=_mesh,
    scratch_types=[
        pltpu.VMEM((CH,), jnp.int32)] * 2 + [pltpu.VMEM((CH,), jnp.float32)] * 12 + [
        pltpu.VMEM((CH, 8), jnp.float32)] * 2 + [
        pltpu.VMEM_SHARED((SP, 8), jnp.float32),
        pltpu.SemaphoreType.DMA,
        pltpu.SemaphoreType.DMA,
        pltpu.SemaphoreType.DMA,
    ],
    compiler_params=_sc_params,
)
def _accum(ev0, ev1, ev2, er0, er1, er2, idx_hbm, zeros_hbm, pout_hbm,
           idx_a, idx_b, a0, a1, a2, a3, a4, a5, c0, c1, c2, c3, c4, c5,
           aug_a, aug_b, sh, sem_a, sem_b, sem_sc):
    c = lax.axis_index("c")
    s = lax.axis_index("s")
    w = c * 16 + s
    lanes = lax.iota(jnp.int32, 16)
    pltpu.sync_copy(zeros_hbm.at[pl.ds(s * STRIPE, STRIPE)],
                    sh.at[pl.ds(s * STRIPE, STRIPE)])
    for aug in (aug_a, aug_b):
        @pl.loop(0, CH // 16)
        def _(g):
            r = g * 16 + lanes
            plsc.store_scatter(aug, [r, _col(6)], jnp.full((16,), 1.0, jnp.float32))
            plsc.store_scatter(aug, [r, _col(7)], jnp.full((16,), 0.0, jnp.float32))

    plsc.subcore_barrier()
    base, cnt = _worker_span(w)
    bufs_a = (a0, a1, a2, a3, a4, a5)
    bufs_b = (c0, c1, c2, c3, c4, c5)
    srcs = (ev0, ev1, ev2, er0, er1, er2)

    def _assemble(bufs, aug):
        @pl.loop(0, CH // 16)
        def _(g):
            r = g * 16 + lanes
            sl = pl.ds(g * 16, 16)
            for col, b in enumerate(bufs):
                plsc.store_scatter(aug, [r, _col(col)], b[sl])

    @pl.loop(base, base + cnt)
    def _(u):
        row_a = (2 * u) * CH
        row_b = row_a + CH
        sl_a = pl.ds(row_a, CH)
        sl_b = pl.ds(row_b, CH)
        da = [pltpu.async_copy(idx_hbm.at[sl_a], idx_a, sem_a)]
        da += [pltpu.async_copy(src.at[sl_a], b, sem_a)
               for b, src in zip(bufs_a, srcs)]
        db = [pltpu.async_copy(idx_hbm.at[sl_b], idx_b, sem_b)]
        db += [pltpu.async_copy(src.at[sl_b], b, sem_b)
               for b, src in zip(bufs_b, srcs)]
        for d in da:
            d.wait()
        _assemble(bufs_a, aug_a)
        scs = []
        for k in range(K):
            sl_k = pl.ds(k * SUB, SUB)
            scs.append(pltpu.async_copy(aug_a.at[sl_k],
                                        sh.at[idx_a.at[sl_k]],
                                        sem_sc, add=True))
        for d in db:
            d.wait()
        _assemble(bufs_b, aug_b)
        for k in range(K):
            sl_k = pl.ds(k * SUB, SUB)
            scs.append(pltpu.async_copy(aug_b.at[sl_k],
                                        sh.at[idx_b.at[sl_k]],
                                        sem_sc, add=True))
        for d in scs:
            d.wait()

    plsc.subcore_barrier()
    pltpu.sync_copy(sh.at[pl.ds(s * STRIPE, STRIPE)],
                    pout_hbm.at[c].at[pl.ds(s * STRIPE, STRIPE)])


def _merge_body(p_ref, o_ref):
    p = p_ref[...]
    sm = p[0] + p[1]
    cnt = jnp.clip(sm[:, 6:7], 1.0, None)
    o_ref[...] = sm / cnt


def _merge(partials):
    bs = 3128
    return pl.pallas_call(
        _merge_body,
        grid=(SP // bs,),
        in_specs=[pl.BlockSpec((2, bs, 8), lambda i: (0, i, 0))],
        out_specs=pl.BlockSpec((bs, 8), lambda i: (i, 0)),
        out_shape=jax.ShapeDtypeStruct((SP, 8), jnp.float32),
    )(partials)


@functools.partial(
    pl.kernel,
    out_type=[jax.ShapeDtypeStruct((N,), jnp.float32)] * 6,
    mesh=_mesh,
    scratch_types=[
        pltpu.VMEM((CH,), jnp.int32)] * 2 + [pltpu.VMEM((CH,), jnp.float32)] * 12 + [
        pltpu.VMEM((CH, 8), jnp.float32)] * 2 + [
        pltpu.SemaphoreType.DMA,
        pltpu.SemaphoreType.DMA,
        pltpu.SemaphoreType.DMA,
        pltpu.SemaphoreType.DMA,
        pltpu.SemaphoreType.DMA,
    ],
    compiler_params=_sc_params,
)
def _center(table, idx_hbm, ev0, ev1, ev2, er0, er1, er2,
            oev0, oev1, oev2, oer0, oer1, oer2,
            idx_a, idx_b, a0, a1, a2, a3, a4, a5, c0, c1, c2, c3, c4, c5,
            rows_a, rows_b, sem_a, sem_b, sem_ga, sem_gb, sem_out):
    c = lax.axis_index("c")
    s = lax.axis_index("s")
    w = c * 16 + s
    lanes = lax.iota(jnp.int32, 16)
    base, cnt = _worker_span(w)
    bufs_a = (a0, a1, a2, a3, a4, a5)
    bufs_b = (c0, c1, c2, c3, c4, c5)
    srcs = (ev0, ev1, ev2, er0, er1, er2)
    outs = (oev0, oev1, oev2, oer0, oer1, oer2)

    def _subtract(bufs, rows):
        @pl.loop(0, CH // 16)
        def _(g):
            r = g * 16 + lanes
            sl = pl.ds(g * 16, 16)
            for col, b in enumerate(bufs):
                b[sl] = b[sl] - plsc.load_gather(rows, [r, _col(col)])

    @pl.loop(base, base + cnt)
    def _(u):
        row_a = (2 * u) * CH
        row_b = row_a + CH
        sl_a = pl.ds(row_a, CH)
        sl_b = pl.ds(row_b, CH)
        dia = pltpu.async_copy(idx_hbm.at[sl_a], idx_a, sem_a)
        dib = pltpu.async_copy(idx_hbm.at[sl_b], idx_b, sem_b)
        da = [pltpu.async_copy(src.at[sl_a], b, sem_a)
              for b, src in zip(bufs_a, srcs)]
        db = [pltpu.async_copy(src.at[sl_b], b, sem_b)
              for b, src in zip(bufs_b, srcs)]
        dia.wait()
        ga = [pltpu.async_copy(table.at[idx_a.at[pl.ds(k * SUB, SUB)]],
                               rows_a.at[pl.ds(k * SUB, SUB)], sem_ga)
              for k in range(K)]
        dib.wait()
        gb = [pltpu.async_copy(table.at[idx_b.at[pl.ds(k * SUB, SUB)]],
                               rows_b.at[pl.ds(k * SUB, SUB)], sem_gb)
              for k in range(K)]
        for d in da:
            d.wait()
        for d in ga:
            d.wait()
        _subtract(bufs_a, rows_a)
        oda = [pltpu.async_copy(b, dst.at[sl_a], sem_out)
               for b, dst in zip(bufs_a, outs)]
        for d in db:
            d.wait()
        for d in gb:
            d.wait()
        _subtract(bufs_b, rows_b)
        odb = [pltpu.async_copy(b, dst.at[sl_b], sem_out)
               for b, dst in zip(bufs_b, outs)]
        for d in oda:
            d.wait()
        for d in odb:
            d.wait()


_GRID = 25
_BLK = N // _GRID


def _wrap_pi(x):
    return jnp.remainder(x + PI, TWO_PI) - PI


def _dense_body(t_ref, f00, f01, f02, v00, v01, v02, e0, e1, e2, r0, r1, r2,
                ft0, ft1, ft2, vt0, vt1, vt2, rt0, rt1, rt2):
    tt = 2.0 * t_ref[...]
    e = jnp.exp(-tt)
    sigma_v = jnp.sqrt(jnp.clip(1.0 - e * e, EPS, None))
    coeff = (1.0 - e) / (1.0 + e)
    sig_r = jnp.sqrt(jnp.clip(2.0 * tt + 8.0 / (1.0 + jnp.exp(tt)) - 4.0, EPS, None))
    for f0c, v0c, evc, erc, ftc, vtc, rtc in (
            (f00, v00, e0, r0, ft0, vt0, rt0),
            (f01, v01, e1, r1, ft1, vt1, rt1),
            (f02, v02, e2, r2, ft2, vt2, rt2)):
        v0i = TWO_PI * v0c[...]
        f0i = TWO_PI * (jnp.remainder(f0c[...] + 0.5, 1.0) - 0.5)
        v_t = e * v0i + sigma_v * evc[...]
        mu = _wrap_pi(coeff * (v_t + v0i))
        r_t = _wrap_pi(mu + sig_r * erc[...])
        f_t = _wrap_pi(f0i + r_t)
        ftc[...] = f_t * INV_TWO_PI
        vtc[...] = v_t * INV_TWO_PI
        rtc[...] = r_t * INV_TWO_PI


def _dense(t, planes):
    spec = pl.BlockSpec((_BLK,), lambda i: (i,))
    return pl.pallas_call(
        _dense_body,
        grid=(_GRID,),
        in_specs=[spec] * 13,
        out_specs=[spec] * 9,
        out_shape=[jax.ShapeDtypeStruct((N,), jnp.float32)] * 9,
    )(t, *planes)


def kernel(t, f0, index, v0, epsilon_v, epsilon_r):
    evp = [epsilon_v[:, i] for i in range(3)]
    erp = [epsilon_r[:, i] for i in range(3)]
    f0p = [f0[:, i] for i in range(3)]
    v0p = [v0[:, i] for i in range(3)]
    zeros = jnp.zeros((SP, 8), jnp.float32)
    partials = _accum(*evp, *erp, index, zeros)
    table = _merge(partials)
    cent = _center(table, index, *evp, *erp)
    outs = _dense(t, f0p + v0p + list(cent))
    ft = jnp.stack(outs[0:3], axis=1)
    vt = jnp.stack(outs[3:6], axis=1)
    rt = jnp.stack(outs[6:9], axis=1)
    evc = jnp.stack(cent[0:3], axis=1)
    erc = jnp.stack(cent[3:6], axis=1)
    return (ft, vt, evc, erc, rt)
